# baseline (speedup 1.0000x reference)
import functools

import jax
import jax.numpy as jnp
from jax import lax
from jax.experimental import pallas as pl
from jax.experimental.pallas import tpu as pltpu
from jax.experimental.pallas import tpu_sc as plsc

NC = 2
NS = 16
NW = NC * NS
CHUNK = 64


def _make_sc_kernel(n, e_pad, p_pad, item_dim, op_dim):
    niter_e = e_pad // CHUNK // NW
    niter_p = p_pad // CHUNK // NW
    n_acc = n + 8
    ZROWS = 400
    n_zchunks = n // ZROWS
    n_ziter = n_zchunks // NS + 1
    KQ = 8
    KR = 2

    mesh = plsc.VectorSubcoreMesh(core_axis_name="c", subcore_axis_name="s",
                                  num_cores=NC, num_subcores=NS)

    def body(items_hbm, ops_hbm, parents_hbm, iedge_hbm, oedge_hbm,
             zitems_hbm, zops_hbm,
             par_out, accc_out, acco_out,
             acc_items, acc_ops,
             si_ring, di_ring, rows, oprows, sem_i, sem_g, sem_s):
        cid = lax.axis_index("c")
        sid = lax.axis_index("s")
        wid = sid * NC + cid

        def zbody(k, _):
            c = sid + NS * k

            @pl.when(c < n_zchunks)
            def _():
                r0 = c * ZROWS
                pltpu.sync_copy(zitems_hbm.at[pl.ds(r0, ZROWS)],
                                acc_items.at[pl.ds(r0, ZROWS)])
                pltpu.sync_copy(zops_hbm.at[pl.ds(r0, ZROWS)],
                                acc_ops.at[pl.ds(r0, ZROWS)])
            return ()

        lax.fori_loop(0, n_ziter, zbody, (), unroll=False)
        plsc.subcore_barrier()

        def run_edges(edge_hbm, table_hbm, acc, ring, niter):
            def start_idx(k):
                c = wid + NW * k
                q = lax.rem(k, KQ)
                pltpu.async_copy(edge_hbm.at[1, c], si_ring.at[q], sem_i)
                pltpu.async_copy(edge_hbm.at[0, c], di_ring.at[q], sem_i)

            def wait_idx():
                pltpu.make_async_copy(edge_hbm.at[1, 0], si_ring.at[0],
                                      sem_i).wait()
                pltpu.make_async_copy(edge_hbm.at[0, 0], di_ring.at[0],
                                      sem_i).wait()

            def start_gather(k):
                q = lax.rem(k, KQ)
                b = lax.rem(k, KR)
                pltpu.async_copy(table_hbm.at[si_ring.at[q]], ring.at[b],
                                 sem_g)

            def wait_gather():
                pltpu.make_async_copy(table_hbm.at[si_ring.at[0]], ring.at[0],
                                      sem_g).wait()

            def start_scatter(k):
                q = lax.rem(k, KQ)
                b = lax.rem(k, KR)
                pltpu.async_copy(ring.at[b], acc.at[di_ring.at[q]], sem_s,
                                 add=True)

            def wait_scatter():
                pltpu.make_async_copy(ring.at[0], acc.at[di_ring.at[0]],
                                      sem_s).wait()

            for q in range(KQ):
                start_idx(q)

            def lbody(j, _):
                wait_idx()

                @pl.when(j >= KR)
                def _():
                    wait_scatter()

                @pl.when(jnp.logical_and(j >= KR, j - KR + KQ < niter))
                def _():
                    start_idx(j - KR + KQ)
                start_gather(j)

                @pl.when(j > 0)
                def _():
                    wait_gather()
                    start_scatter(j - 1)
                return ()

            lax.fori_loop(0, niter, lbody, (), unroll=False)
            wait_gather()
            start_scatter(niter - 1)
            for _ in range(min(KR, niter)):
                wait_scatter()

        run_edges(iedge_hbm, items_hbm, acc_items, rows, niter_e)
        run_edges(oedge_hbm, ops_hbm, acc_ops, oprows, niter_e)

        def pbody(k, _):
            c = wid + NW * k
            pltpu.sync_copy(parents_hbm.at[pl.ds(c * CHUNK, CHUNK)],
                            si_ring.at[0])
            pltpu.async_copy(items_hbm.at[si_ring.at[0]], rows.at[0],
                             sem_g).wait()
            pltpu.sync_copy(rows.at[0], par_out.at[pl.ds(c * CHUNK, CHUNK)])
            return ()

        lax.fori_loop(0, niter_p, pbody, (), unroll=False)

        plsc.subcore_barrier()

        def wbody(k, _):
            c = sid + NS * k

            @pl.when(c < n_zchunks)
            def _():
                r0 = c * ZROWS
                pltpu.sync_copy(acc_items.at[pl.ds(r0, ZROWS)],
                                accc_out.at[cid, pl.ds(r0, ZROWS)])
                pltpu.sync_copy(acc_ops.at[pl.ds(r0, ZROWS)],
                                acco_out.at[cid, pl.ds(r0, ZROWS)])
            return ()

        lax.fori_loop(0, n_ziter, wbody, (), unroll=False)

    return pl.kernel(
        body,
        out_type=(
            jax.ShapeDtypeStruct((p_pad, item_dim), jnp.float32),
            jax.ShapeDtypeStruct((NC, n, item_dim), jnp.float32),
            jax.ShapeDtypeStruct((NC, n, op_dim), jnp.float32),
        ),
        mesh=mesh,
        compiler_params=pltpu.CompilerParams(use_tc_tiling_on_sc=False),
        scratch_types=[
            pltpu.VMEM_SHARED((n_acc, item_dim), jnp.float32),
            pltpu.VMEM_SHARED((n_acc, op_dim), jnp.float32),
            pltpu.VMEM((KQ, CHUNK), jnp.int32),
            pltpu.VMEM((KQ, CHUNK), jnp.int32),
            pltpu.VMEM((KR, CHUNK, item_dim), jnp.float32),
            pltpu.VMEM((KR, CHUNK, op_dim), jnp.float32),
            pltpu.SemaphoreType.DMA,
            pltpu.SemaphoreType.DMA,
            pltpu.SemaphoreType.DMA,
        ],
    )


def _tc_body(n, blk, items_ref, par_ref, accc_ref, acco_ref,
             Ws1, bs1, Ws2, bs2, Wp1, bp1, Wp2, bp2, Wc1, bc1, Wc2, bc2,
             Wo1, bo1, Wo2, bo2, Wm1, bm1, Wm2, bm2, Wm3, bm3, out_ref):
    prec = lax.Precision.HIGHEST

    def mlp2(x, W1, b1, W2, b2):
        h = jnp.maximum(jnp.dot(x, W1[...], precision=prec) + b1[...], 0.0)
        return jnp.dot(h, W2[...], precision=prec) + b2[...]

    self_emb = mlp2(items_ref[...], Ws1, bs1, Ws2, bs2)
    parent_emb = mlp2(par_ref[...], Wp1, bp1, Wp2, bp2)
    child_in = accc_ref[0] + accc_ref[1]
    child_emb = mlp2(child_in, Wc1, bc1, Wc2, bc2)
    ops_in = acco_ref[0] + acco_ref[1]
    ops_emb = mlp2(ops_in, Wo1, bo1, Wo2, bo2)

    comb = jnp.concatenate([parent_emb, child_emb, ops_emb, self_emb], axis=-1)
    h = jnp.maximum(jnp.dot(comb, Wm1[...], precision=prec) + bm1[...], 0.0)
    h = jnp.maximum(jnp.dot(h, Wm2[...], precision=prec) + bm2[...], 0.0)
    h = jnp.dot(h, Wm3[...], precision=prec) + bm3[...]

    i = pl.program_id(0)
    gid = i * blk + lax.broadcasted_iota(jnp.int32, h.shape, 0)
    out_ref[...] = jnp.where(gid == n - 1, 0.0, h)


def kernel(items, parents, operations, item_edge_index, op_edge_index,
           Ws1, bs1, Ws2, bs2, Wp1, bp1, Wp2, bp2, Wc1, bc1, Wc2, bc2,
           Wo1, bo1, Wo2, bo2, Wm1, bm1, Wm2, bm2, Wm3, bm3):
    n, item_dim = items.shape
    op_dim = operations.shape[1]
    e = item_edge_index.shape[1]
    out_dim = Wm3.shape[1]

    grain = CHUNK * NW
    e_pad = -(-e // grain) * grain
    p_pad = -(-n // grain) * grain

    def pad_edges(eidx):
        eidx = eidx.astype(jnp.int32)
        dst = jnp.pad(eidx[0], (0, e_pad - e), constant_values=n)
        src = jnp.pad(eidx[1], (0, e_pad - e), constant_values=0)
        return jnp.stack([dst, src]).reshape(2, e_pad // CHUNK, CHUNK)

    parents32 = jnp.pad(parents.astype(jnp.int32), (0, p_pad - n))
    iedge = pad_edges(item_edge_index)
    oedge = pad_edges(op_edge_index)
    zitems = jnp.zeros((n, item_dim), jnp.float32)
    zops = jnp.zeros((n, op_dim), jnp.float32)

    sc = _make_sc_kernel(n, e_pad, p_pad, item_dim, op_dim)
    par_rows, accc, acco = sc(items, operations, parents32, iedge, oedge,
                              zitems, zops)

    blk = 1000
    grid = n // blk
    full = lambda shape: pl.BlockSpec(shape, lambda i: (0,) * len(shape))
    w_specs = [full(w.shape) for w in
               (Ws1, bs1, Ws2, bs2, Wp1, bp1, Wp2, bp2, Wc1, bc1, Wc2, bc2,
                Wo1, bo1, Wo2, bo2, Wm1, bm1, Wm2, bm2, Wm3, bm3)]

    out = pl.pallas_call(
        functools.partial(_tc_body, n, blk),
        grid=(grid,),
        in_specs=[
            pl.BlockSpec((blk, item_dim), lambda i: (i, 0)),
            pl.BlockSpec((blk, item_dim), lambda i: (i, 0)),
            pl.BlockSpec((NC, blk, item_dim), lambda i: (0, i, 0)),
            pl.BlockSpec((NC, blk, op_dim), lambda i: (0, i, 0)),
        ] + w_specs,
        out_specs=pl.BlockSpec((blk, out_dim), lambda i: (i, 0)),
        out_shape=jax.ShapeDtypeStruct((n, out_dim), jnp.float32),
    )(items, par_rows, accc, acco,
      Ws1, bs1, Ws2, bs2, Wp1, bp1, Wp2, bp2, Wc1, bc1, Wc2, bc2,
      Wo1, bo1, Wo2, bo2, Wm1, bm1, Wm2, bm2, Wm3, bm3)
    return out


# speedup vs baseline: 1.6971x; 1.6971x over previous
import functools

import jax
import jax.numpy as jnp
from jax import lax
from jax.experimental import pallas as pl
from jax.experimental.pallas import tpu as pltpu
from jax.experimental.pallas import tpu_sc as plsc

NC = 2
NS = 16
NW = NC * NS
CHUNK = 64


def _make_sc_kernel(n, e, item_dim, op_dim):
    ec = e // CHUNK
    niter_e = -(-ec // NW)
    pc = n // CHUNK
    ptail = n - pc * CHUNK
    niter_p = -(-pc // NW)
    n_acc = n + 8
    ZROWS = 400
    n_zchunks = n // ZROWS
    n_ziter = n_zchunks // NS + 1
    KQ = 8
    KR = 2

    mesh = plsc.VectorSubcoreMesh(core_axis_name="c", subcore_axis_name="s",
                                  num_cores=NC, num_subcores=NS)

    def body(items_hbm, ops_hbm, parents_hbm, iedge_hbm, oedge_hbm,
             zitems_hbm, zops_hbm,
             par_out, accc_out, acco_out,
             acc_items, acc_ops,
             si_ring, di_ring, rows, oprows, pidx_t, sem_i, sem_g, sem_s):
        cid = lax.axis_index("c")
        sid = lax.axis_index("s")
        wid = sid * NC + cid

        def zbody(k, _):
            c = sid + NS * k

            @pl.when(c < n_zchunks)
            def _():
                r0 = c * ZROWS
                pltpu.sync_copy(zitems_hbm.at[pl.ds(r0, ZROWS)],
                                acc_items.at[pl.ds(r0, ZROWS)])
                pltpu.sync_copy(zops_hbm.at[pl.ds(r0, ZROWS)],
                                acc_ops.at[pl.ds(r0, ZROWS)])
            return ()

        lax.fori_loop(0, n_ziter, zbody, (), unroll=False)
        plsc.subcore_barrier()

        def run_edges(edge_hbm, table_hbm, acc, ring, niter, total):
            def cond(k):
                return wid + NW * k < total

            def start_idx(k):
                base = (wid + NW * k) * CHUNK
                q = lax.rem(k, KQ)
                pltpu.async_copy(edge_hbm.at[1, pl.ds(base, CHUNK)],
                                 si_ring.at[q], sem_i)
                pltpu.async_copy(edge_hbm.at[0, pl.ds(base, CHUNK)],
                                 di_ring.at[q], sem_i)

            def wait_idx():
                pltpu.make_async_copy(edge_hbm.at[1, pl.ds(0, CHUNK)],
                                      si_ring.at[0], sem_i).wait()
                pltpu.make_async_copy(edge_hbm.at[0, pl.ds(0, CHUNK)],
                                      di_ring.at[0], sem_i).wait()

            def start_gather(k):
                q = lax.rem(k, KQ)
                b = lax.rem(k, KR)
                pltpu.async_copy(table_hbm.at[si_ring.at[q]], ring.at[b],
                                 sem_g)

            def wait_gather():
                pltpu.make_async_copy(table_hbm.at[si_ring.at[0]], ring.at[0],
                                      sem_g).wait()

            def start_scatter(k):
                q = lax.rem(k, KQ)
                b = lax.rem(k, KR)
                pltpu.async_copy(ring.at[b], acc.at[di_ring.at[q]], sem_s,
                                 add=True)

            def wait_scatter():
                pltpu.make_async_copy(ring.at[0], acc.at[di_ring.at[0]],
                                      sem_s).wait()

            for q in range(KQ):
                start_idx(q)

            def lbody(j, _):
                @pl.when(cond(j))
                def _():
                    wait_idx()

                @pl.when(jnp.logical_and(j >= KR, cond(j - KR)))
                def _():
                    wait_scatter()

                @pl.when(jnp.logical_and(
                    j >= KR, jnp.logical_and(j - KR + KQ < niter,
                                             cond(j - KR + KQ))))
                def _():
                    start_idx(j - KR + KQ)

                @pl.when(cond(j))
                def _():
                    start_gather(j)

                @pl.when(jnp.logical_and(j > 0, cond(j - 1)))
                def _():
                    wait_gather()
                    start_scatter(j - 1)
                return ()

            lax.fori_loop(0, niter, lbody, (), unroll=False)

            @pl.when(cond(niter - 1))
            def _():
                wait_gather()
                start_scatter(niter - 1)

            for t in range(niter - KR, niter):
                @pl.when(cond(t))
                def _():
                    wait_scatter()

        run_edges(iedge_hbm, items_hbm, acc_items, rows, niter_e, ec)
        run_edges(oedge_hbm, ops_hbm, acc_ops, oprows, niter_e, ec)

        def pbody(k, _):
            c = wid + NW * k

            @pl.when(c < pc)
            def _():
                base = c * CHUNK
                pltpu.sync_copy(parents_hbm.at[pl.ds(base, CHUNK)],
                                si_ring.at[0])
                pltpu.async_copy(items_hbm.at[si_ring.at[0]], rows.at[0],
                                 sem_g).wait()
                pltpu.sync_copy(rows.at[0],
                                par_out.at[pl.ds(base, CHUNK)])
            return ()

        lax.fori_loop(0, niter_p, pbody, (), unroll=False)

        if ptail:
            @pl.when(wid == 0)
            def _():
                base = pc * CHUNK
                pltpu.sync_copy(parents_hbm.at[pl.ds(base, ptail)], pidx_t)
                pltpu.async_copy(items_hbm.at[pidx_t],
                                 rows.at[0, pl.ds(0, ptail)], sem_g).wait()
                pltpu.sync_copy(rows.at[0, pl.ds(0, ptail)],
                                par_out.at[pl.ds(base, ptail)])

        plsc.subcore_barrier()

        def wbody(k, _):
            c = sid + NS * k

            @pl.when(c < n_zchunks)
            def _():
                r0 = c * ZROWS
                pltpu.sync_copy(acc_items.at[pl.ds(r0, ZROWS)],
                                accc_out.at[cid, pl.ds(r0, ZROWS)])
                pltpu.sync_copy(acc_ops.at[pl.ds(r0, ZROWS)],
                                acco_out.at[cid, pl.ds(r0, ZROWS)])
            return ()

        lax.fori_loop(0, n_ziter, wbody, (), unroll=False)

    return pl.kernel(
        body,
        out_type=(
            jax.ShapeDtypeStruct((n, item_dim), jnp.float32),
            jax.ShapeDtypeStruct((NC, n, item_dim), jnp.float32),
            jax.ShapeDtypeStruct((NC, n, op_dim), jnp.float32),
        ),
        mesh=mesh,
        compiler_params=pltpu.CompilerParams(use_tc_tiling_on_sc=False),
        scratch_types=[
            pltpu.VMEM_SHARED((n_acc, item_dim), jnp.float32),
            pltpu.VMEM_SHARED((n_acc, op_dim), jnp.float32),
            pltpu.VMEM((KQ, CHUNK), jnp.int32),
            pltpu.VMEM((KQ, CHUNK), jnp.int32),
            pltpu.VMEM((KR, CHUNK, item_dim), jnp.float32),
            pltpu.VMEM((KR, CHUNK, op_dim), jnp.float32),
            pltpu.VMEM((16,), jnp.int32),
            pltpu.SemaphoreType.DMA,
            pltpu.SemaphoreType.DMA,
            pltpu.SemaphoreType.DMA,
        ],
    )


def _tc_body(n, blk, items_ref, par_ref, accc_ref, acco_ref,
             Ws1, bs1, Ws2, bs2, Wp1, bp1, Wp2, bp2, Wc1, bc1, Wc2, bc2,
             Wo1, bo1, Wo2, bo2, Wm1, bm1, Wm2, bm2, Wm3, bm3, out_ref):
    def mlp2(x, W1, b1, W2, b2):
        h = jnp.maximum(jnp.dot(x, W1[...]) + b1[...], 0.0)
        return jnp.dot(h, W2[...]) + b2[...]

    self_emb = mlp2(items_ref[...], Ws1, bs1, Ws2, bs2)
    parent_emb = mlp2(par_ref[...], Wp1, bp1, Wp2, bp2)
    child_in = accc_ref[0] + accc_ref[1]
    child_emb = mlp2(child_in, Wc1, bc1, Wc2, bc2)
    ops_in = acco_ref[0] + acco_ref[1]
    ops_emb = mlp2(ops_in, Wo1, bo1, Wo2, bo2)

    comb = jnp.concatenate([parent_emb, child_emb, ops_emb, self_emb], axis=-1)
    h = jnp.maximum(jnp.dot(comb, Wm1[...]) + bm1[...], 0.0)
    h = jnp.maximum(jnp.dot(h, Wm2[...]) + bm2[...], 0.0)
    h = jnp.dot(h, Wm3[...]) + bm3[...]

    i = pl.program_id(0)
    gid = i * blk + lax.broadcasted_iota(jnp.int32, h.shape, 0)
    out_ref[...] = jnp.where(gid == n - 1, 0.0, h)


def kernel(items, parents, operations, item_edge_index, op_edge_index,
           Ws1, bs1, Ws2, bs2, Wp1, bp1, Wp2, bp2, Wc1, bc1, Wc2, bc2,
           Wo1, bo1, Wo2, bo2, Wm1, bm1, Wm2, bm2, Wm3, bm3):
    n, item_dim = items.shape
    op_dim = operations.shape[1]
    e = item_edge_index.shape[1]
    out_dim = Wm3.shape[1]

    parents32 = parents.astype(jnp.int32)
    iedge = item_edge_index.astype(jnp.int32)
    oedge = op_edge_index.astype(jnp.int32)
    zitems = jnp.zeros((n, item_dim), jnp.float32)
    zops = jnp.zeros((n, op_dim), jnp.float32)

    sc = _make_sc_kernel(n, e, item_dim, op_dim)
    par_rows, accc, acco = sc(items, operations, parents32, iedge, oedge,
                              zitems, zops)

    blk = 1000
    grid = n // blk
    full = lambda shape: pl.BlockSpec(shape, lambda i: (0,) * len(shape))
    w_specs = [full(w.shape) for w in
               (Ws1, bs1, Ws2, bs2, Wp1, bp1, Wp2, bp2, Wc1, bc1, Wc2, bc2,
                Wo1, bo1, Wo2, bo2, Wm1, bm1, Wm2, bm2, Wm3, bm3)]

    out = pl.pallas_call(
        functools.partial(_tc_body, n, blk),
        grid=(grid,),
        in_specs=[
            pl.BlockSpec((blk, item_dim), lambda i: (i, 0)),
            pl.BlockSpec((blk, item_dim), lambda i: (i, 0)),
            pl.BlockSpec((NC, blk, item_dim), lambda i: (0, i, 0)),
            pl.BlockSpec((NC, blk, op_dim), lambda i: (0, i, 0)),
        ] + w_specs,
        out_specs=pl.BlockSpec((blk, out_dim), lambda i: (i, 0)),
        out_shape=jax.ShapeDtypeStruct((n, out_dim), jnp.float32),
    )(items, par_rows, accc, acco,
      Ws1, bs1, Ws2, bs2, Wp1, bp1, Wp2, bp2, Wc1, bc1, Wc2, bc2,
      Wo1, bo1, Wo2, bo2, Wm1, bm1, Wm2, bm2, Wm3, bm3)
    return out


# speedup vs baseline: 2.1707x; 1.2790x over previous
import functools

import jax
import jax.numpy as jnp
from jax import lax
from jax.experimental import pallas as pl
from jax.experimental.pallas import tpu as pltpu
from jax.experimental.pallas import tpu_sc as plsc

NC = 2
NS = 16
NW = NC * NS
CHUNK = 64


def _make_sc_kernel(n, e, item_dim, op_dim):
    ec = e // CHUNK
    niter_e = -(-ec // NW)
    pc = n // CHUNK
    ptail = n - pc * CHUNK
    niter_p = -(-pc // NW)
    n_acc = n + 8
    ZROWS = 400
    n_zchunks = n // ZROWS
    n_ziter = n_zchunks // NS + 1
    KQ = 8
    KR = 2

    mesh = plsc.VectorSubcoreMesh(core_axis_name="c", subcore_axis_name="s",
                                  num_cores=NC, num_subcores=NS)

    def body(items_hbm, ops_hbm, parents_hbm, iedge_hbm, oedge_hbm,
             zitems_hbm, zops_hbm,
             par_out, accc_out, acco_out,
             acc_items, acc_ops,
             si_ring, di_ring, si2_ring, di2_ring, rows, oprows, pidx_t,
             sem_i, sem_g, sem_s, sem_i2, sem_g2, sem_s2):
        cid = lax.axis_index("c")
        sid = lax.axis_index("s")
        wid = sid * NC + cid

        def zbody(k, _):
            c = sid + NS * k

            @pl.when(c < n_zchunks)
            def _():
                r0 = c * ZROWS
                pltpu.sync_copy(zitems_hbm.at[pl.ds(r0, ZROWS)],
                                acc_items.at[pl.ds(r0, ZROWS)])
                pltpu.sync_copy(zops_hbm.at[pl.ds(r0, ZROWS)],
                                acc_ops.at[pl.ds(r0, ZROWS)])
            return ()

        lax.fori_loop(0, n_ziter, zbody, (), unroll=False)
        plsc.subcore_barrier()

        def make_stream(edge_hbm, table_hbm, acc, ring, si_r, di_r,
                        s_i, s_g, s_s):
            def cond(k):
                return wid + NW * k < ec

            def start_idx(k):
                base = (wid + NW * k) * CHUNK
                q = lax.rem(k, KQ)
                pltpu.async_copy(edge_hbm.at[1, pl.ds(base, CHUNK)],
                                 si_r.at[q], s_i)
                pltpu.async_copy(edge_hbm.at[0, pl.ds(base, CHUNK)],
                                 di_r.at[q], s_i)

            def wait_idx():
                pltpu.make_async_copy(edge_hbm.at[1, pl.ds(0, CHUNK)],
                                      si_r.at[0], s_i).wait()
                pltpu.make_async_copy(edge_hbm.at[0, pl.ds(0, CHUNK)],
                                      di_r.at[0], s_i).wait()

            def start_gather(k):
                q = lax.rem(k, KQ)
                b = lax.rem(k, KR)
                pltpu.async_copy(table_hbm.at[si_r.at[q]], ring.at[b], s_g)

            def wait_gather():
                pltpu.make_async_copy(table_hbm.at[si_r.at[0]], ring.at[0],
                                      s_g).wait()

            def start_scatter(k):
                q = lax.rem(k, KQ)
                b = lax.rem(k, KR)
                pltpu.async_copy(ring.at[b], acc.at[di_r.at[q]], s_s,
                                 add=True)

            def wait_scatter():
                pltpu.make_async_copy(ring.at[0], acc.at[di_r.at[0]],
                                      s_s).wait()

            return cond, start_idx, wait_idx, start_gather, wait_gather, \
                start_scatter, wait_scatter

        icond, i_sidx, i_widx, i_sg, i_wg, i_ss, i_ws = make_stream(
            iedge_hbm, items_hbm, acc_items, rows, si_ring, di_ring,
            sem_i, sem_g, sem_s)
        ocond, o_sidx, o_widx, o_sg, o_wg, o_ss, o_ws = make_stream(
            oedge_hbm, ops_hbm, acc_ops, oprows, si2_ring, di2_ring,
            sem_i2, sem_g2, sem_s2)

        for q in range(KQ):
            i_sidx(q)
            o_sidx(q)

        def lbody(j, _):
            @pl.when(icond(j))
            def _():
                i_widx()
                o_widx()

            @pl.when(jnp.logical_and(j >= KR, icond(j - KR)))
            def _():
                i_ws()
                o_ws()

            @pl.when(jnp.logical_and(
                j >= KR, jnp.logical_and(j - KR + KQ < niter_e,
                                         icond(j - KR + KQ))))
            def _():
                i_sidx(j - KR + KQ)
                o_sidx(j - KR + KQ)

            @pl.when(icond(j))
            def _():
                i_sg(j)
                o_sg(j)

            @pl.when(jnp.logical_and(j > 0, icond(j - 1)))
            def _():
                i_wg()
                i_ss(j - 1)
                o_wg()
                o_ss(j - 1)
            return ()

        lax.fori_loop(0, niter_e, lbody, (), unroll=False)

        @pl.when(icond(niter_e - 1))
        def _():
            i_wg()
            i_ss(niter_e - 1)
            o_wg()
            o_ss(niter_e - 1)

        for t in range(niter_e - KR, niter_e):
            @pl.when(icond(t))
            def _():
                i_ws()
                o_ws()

        def pbody(k, _):
            c = wid + NW * k

            @pl.when(c < pc)
            def _():
                base = c * CHUNK
                pltpu.sync_copy(parents_hbm.at[pl.ds(base, CHUNK)],
                                si_ring.at[0])
                pltpu.async_copy(items_hbm.at[si_ring.at[0]], rows.at[0],
                                 sem_g).wait()
                pltpu.sync_copy(rows.at[0],
                                par_out.at[pl.ds(base, CHUNK)])
            return ()

        lax.fori_loop(0, niter_p, pbody, (), unroll=False)

        if ptail:
            @pl.when(wid == 0)
            def _():
                base = pc * CHUNK
                pltpu.sync_copy(parents_hbm.at[pl.ds(base, ptail)], pidx_t)
                pltpu.async_copy(items_hbm.at[pidx_t],
                                 rows.at[0, pl.ds(0, ptail)], sem_g).wait()
                pltpu.sync_copy(rows.at[0, pl.ds(0, ptail)],
                                par_out.at[pl.ds(base, ptail)])

        plsc.subcore_barrier()

        def wbody(k, _):
            c = sid + NS * k

            @pl.when(c < n_zchunks)
            def _():
                r0 = c * ZROWS
                pltpu.sync_copy(acc_items.at[pl.ds(r0, ZROWS)],
                                accc_out.at[cid, pl.ds(r0, ZROWS)])
                pltpu.sync_copy(acc_ops.at[pl.ds(r0, ZROWS)],
                                acco_out.at[cid, pl.ds(r0, ZROWS)])
            return ()

        lax.fori_loop(0, n_ziter, wbody, (), unroll=False)

    return pl.kernel(
        body,
        out_type=(
            jax.ShapeDtypeStruct((n, item_dim), jnp.float32),
            jax.ShapeDtypeStruct((NC, n, item_dim), jnp.float32),
            jax.ShapeDtypeStruct((NC, n, op_dim), jnp.float32),
        ),
        mesh=mesh,
        compiler_params=pltpu.CompilerParams(use_tc_tiling_on_sc=False),
        scratch_types=[
            pltpu.VMEM_SHARED((n_acc, item_dim), jnp.float32),
            pltpu.VMEM_SHARED((n_acc, op_dim), jnp.float32),
            pltpu.VMEM((KQ, CHUNK), jnp.int32),
            pltpu.VMEM((KQ, CHUNK), jnp.int32),
            pltpu.VMEM((KQ, CHUNK), jnp.int32),
            pltpu.VMEM((KQ, CHUNK), jnp.int32),
            pltpu.VMEM((KR, CHUNK, item_dim), jnp.float32),
            pltpu.VMEM((KR, CHUNK, op_dim), jnp.float32),
            pltpu.VMEM((16,), jnp.int32),
            pltpu.SemaphoreType.DMA,
            pltpu.SemaphoreType.DMA,
            pltpu.SemaphoreType.DMA,
            pltpu.SemaphoreType.DMA,
            pltpu.SemaphoreType.DMA,
            pltpu.SemaphoreType.DMA,
        ],
    )


def _tc_body(n, blk, items_ref, par_ref, accc_ref, acco_ref,
             Ws1, bs1, Ws2, bs2, Wp1, bp1, Wp2, bp2, Wc1, bc1, Wc2, bc2,
             Wo1, bo1, Wo2, bo2, Wm1, bm1, Wm2, bm2, Wm3, bm3, out_ref):
    def mlp2(x, W1, b1, W2, b2):
        h = jnp.maximum(jnp.dot(x, W1[...]) + b1[...], 0.0)
        return jnp.dot(h, W2[...]) + b2[...]

    self_emb = mlp2(items_ref[...], Ws1, bs1, Ws2, bs2)
    parent_emb = mlp2(par_ref[...], Wp1, bp1, Wp2, bp2)
    child_in = accc_ref[0] + accc_ref[1]
    child_emb = mlp2(child_in, Wc1, bc1, Wc2, bc2)
    ops_in = acco_ref[0] + acco_ref[1]
    ops_emb = mlp2(ops_in, Wo1, bo1, Wo2, bo2)

    comb = jnp.concatenate([parent_emb, child_emb, ops_emb, self_emb], axis=-1)
    h = jnp.maximum(jnp.dot(comb, Wm1[...]) + bm1[...], 0.0)
    h = jnp.maximum(jnp.dot(h, Wm2[...]) + bm2[...], 0.0)
    h = jnp.dot(h, Wm3[...]) + bm3[...]

    i = pl.program_id(0)
    gid = i * blk + lax.broadcasted_iota(jnp.int32, h.shape, 0)
    out_ref[...] = jnp.where(gid == n - 1, 0.0, h)


def kernel(items, parents, operations, item_edge_index, op_edge_index,
           Ws1, bs1, Ws2, bs2, Wp1, bp1, Wp2, bp2, Wc1, bc1, Wc2, bc2,
           Wo1, bo1, Wo2, bo2, Wm1, bm1, Wm2, bm2, Wm3, bm3):
    n, item_dim = items.shape
    op_dim = operations.shape[1]
    e = item_edge_index.shape[1]
    out_dim = Wm3.shape[1]

    parents32 = parents.astype(jnp.int32)
    iedge = item_edge_index.astype(jnp.int32)
    oedge = op_edge_index.astype(jnp.int32)
    zitems = jnp.zeros((n, item_dim), jnp.float32)
    zops = jnp.zeros((n, op_dim), jnp.float32)

    sc = _make_sc_kernel(n, e, item_dim, op_dim)
    par_rows, accc, acco = sc(items, operations, parents32, iedge, oedge,
                              zitems, zops)

    blk = 1000
    grid = n // blk
    full = lambda shape: pl.BlockSpec(shape, lambda i: (0,) * len(shape))
    w_specs = [full(w.shape) for w in
               (Ws1, bs1, Ws2, bs2, Wp1, bp1, Wp2, bp2, Wc1, bc1, Wc2, bc2,
                Wo1, bo1, Wo2, bo2, Wm1, bm1, Wm2, bm2, Wm3, bm3)]

    out = pl.pallas_call(
        functools.partial(_tc_body, n, blk),
        grid=(grid,),
        in_specs=[
            pl.BlockSpec((blk, item_dim), lambda i: (i, 0)),
            pl.BlockSpec((blk, item_dim), lambda i: (i, 0)),
            pl.BlockSpec((NC, blk, item_dim), lambda i: (0, i, 0)),
            pl.BlockSpec((NC, blk, op_dim), lambda i: (0, i, 0)),
        ] + w_specs,
        out_specs=pl.BlockSpec((blk, out_dim), lambda i: (i, 0)),
        out_shape=jax.ShapeDtypeStruct((n, out_dim), jnp.float32),
    )(items, par_rows, accc, acco,
      Ws1, bs1, Ws2, bs2, Wp1, bp1, Wp2, bp2, Wc1, bc1, Wc2, bc2,
      Wo1, bo1, Wo2, bo2, Wm1, bm1, Wm2, bm2, Wm3, bm3)
    return out


# speedup vs baseline: 2.1781x; 1.0034x over previous
import functools

import jax
import jax.numpy as jnp
from jax import lax
from jax.experimental import pallas as pl
from jax.experimental.pallas import tpu as pltpu
from jax.experimental.pallas import tpu_sc as plsc

NC = 2
NS = 16
NW = NC * NS
CHUNK = 64


def _make_sc_kernel(n, e, item_dim, op_dim):
    ec = e // CHUNK
    niter_e = -(-ec // NW)
    pc = n // CHUNK
    ptail = n - pc * CHUNK
    niter_p = -(-pc // NW)
    n_acc = n + 8
    ZROWS = 400
    n_zchunks = n // ZROWS
    n_ziter = n_zchunks // NS + 1
    KQ = 8
    KR = 2

    mesh = plsc.VectorSubcoreMesh(core_axis_name="c", subcore_axis_name="s",
                                  num_cores=NC, num_subcores=NS)

    def body(items_hbm, ops_hbm, parents_hbm, iedge_hbm, oedge_hbm,
             zitems_hbm, zops_hbm,
             par_out, accc_out, acco_out,
             acc_items, acc_ops,
             si_ring, di_ring, si2_ring, di2_ring, rows, oprows, pidx_t,
             sem_i, sem_g, sem_s, sem_i2, sem_g2, sem_s2):
        cid = lax.axis_index("c")
        sid = lax.axis_index("s")
        wid = sid * NC + cid

        def zbody(k, _):
            c = sid + NS * k

            @pl.when(c < n_zchunks)
            def _():
                r0 = c * ZROWS
                pltpu.sync_copy(zitems_hbm.at[pl.ds(r0, ZROWS)],
                                acc_items.at[pl.ds(r0, ZROWS)])
                pltpu.sync_copy(zops_hbm.at[pl.ds(r0, ZROWS)],
                                acc_ops.at[pl.ds(r0, ZROWS)])
            return ()

        lax.fori_loop(0, n_ziter, zbody, (), unroll=False)
        plsc.subcore_barrier()

        def make_stream(edge_hbm, table_hbm, acc, ring, si_r, di_r,
                        s_i, s_g, s_s):
            def cond(k):
                return wid + NW * k < ec

            def start_idx(k):
                base = (wid + NW * k) * CHUNK
                q = lax.rem(k, KQ)
                pltpu.async_copy(edge_hbm.at[1, pl.ds(base, CHUNK)],
                                 si_r.at[q], s_i)
                pltpu.async_copy(edge_hbm.at[0, pl.ds(base, CHUNK)],
                                 di_r.at[q], s_i)

            def wait_idx():
                pltpu.make_async_copy(edge_hbm.at[1, pl.ds(0, CHUNK)],
                                      si_r.at[0], s_i).wait()
                pltpu.make_async_copy(edge_hbm.at[0, pl.ds(0, CHUNK)],
                                      di_r.at[0], s_i).wait()

            def start_gather(k):
                q = lax.rem(k, KQ)
                b = lax.rem(k, KR)
                pltpu.async_copy(table_hbm.at[si_r.at[q]], ring.at[b], s_g)

            def wait_gather():
                pltpu.make_async_copy(table_hbm.at[si_r.at[0]], ring.at[0],
                                      s_g).wait()

            def start_scatter(k):
                q = lax.rem(k, KQ)
                b = lax.rem(k, KR)
                pltpu.async_copy(ring.at[b], acc.at[di_r.at[q]], s_s,
                                 add=True)

            def wait_scatter():
                pltpu.make_async_copy(ring.at[0], acc.at[di_r.at[0]],
                                      s_s).wait()

            return cond, start_idx, wait_idx, start_gather, wait_gather, \
                start_scatter, wait_scatter

        icond, i_sidx, i_widx, i_sg, i_wg, i_ss, i_ws = make_stream(
            iedge_hbm, items_hbm, acc_items, rows, si_ring, di_ring,
            sem_i, sem_g, sem_s)
        ocond, o_sidx, o_widx, o_sg, o_wg, o_ss, o_ws = make_stream(
            oedge_hbm, ops_hbm, acc_ops, oprows, si2_ring, di2_ring,
            sem_i2, sem_g2, sem_s2)

        for q in range(KQ):
            i_sidx(q)
            o_sidx(q)

        def lbody(j, _):
            @pl.when(icond(j))
            def _():
                i_widx()
                o_widx()

            @pl.when(jnp.logical_and(j >= KR, icond(j - KR)))
            def _():
                i_ws()
                o_ws()

            @pl.when(jnp.logical_and(
                j >= KR, jnp.logical_and(j - KR + KQ < niter_e,
                                         icond(j - KR + KQ))))
            def _():
                i_sidx(j - KR + KQ)
                o_sidx(j - KR + KQ)

            @pl.when(icond(j))
            def _():
                i_sg(j)
                o_sg(j)

            @pl.when(jnp.logical_and(j > 0, icond(j - 1)))
            def _():
                i_wg()
                i_ss(j - 1)
                o_wg()
                o_ss(j - 1)
            return ()

        lax.fori_loop(0, niter_e, lbody, (), unroll=False)

        @pl.when(icond(niter_e - 1))
        def _():
            i_wg()
            i_ss(niter_e - 1)
            o_wg()
            o_ss(niter_e - 1)

        for t in range(niter_e - KR, niter_e):
            @pl.when(icond(t))
            def _():
                i_ws()
                o_ws()

        def pbody(k, _):
            c = wid + NW * k

            @pl.when(c < pc)
            def _():
                base = c * CHUNK
                pltpu.sync_copy(parents_hbm.at[pl.ds(base, CHUNK)],
                                si_ring.at[0])
                pltpu.async_copy(items_hbm.at[si_ring.at[0]], rows.at[0],
                                 sem_g).wait()
                pltpu.sync_copy(rows.at[0],
                                par_out.at[pl.ds(base, CHUNK)])
            return ()

        lax.fori_loop(0, niter_p, pbody, (), unroll=False)

        if ptail:
            @pl.when(wid == 0)
            def _():
                base = pc * CHUNK
                pltpu.sync_copy(parents_hbm.at[pl.ds(base, ptail)], pidx_t)
                pltpu.async_copy(items_hbm.at[pidx_t],
                                 rows.at[0, pl.ds(0, ptail)], sem_g).wait()
                pltpu.sync_copy(rows.at[0, pl.ds(0, ptail)],
                                par_out.at[pl.ds(base, ptail)])

        plsc.subcore_barrier()

        def wbody(k, _):
            c = sid + NS * k

            @pl.when(c < n_zchunks)
            def _():
                r0 = c * ZROWS
                pltpu.sync_copy(acc_items.at[pl.ds(r0, ZROWS)],
                                accc_out.at[cid, pl.ds(r0, ZROWS)])
                pltpu.sync_copy(acc_ops.at[pl.ds(r0, ZROWS)],
                                acco_out.at[cid, pl.ds(r0, ZROWS)])
            return ()

        lax.fori_loop(0, n_ziter, wbody, (), unroll=False)

    return pl.kernel(
        body,
        out_type=(
            jax.ShapeDtypeStruct((n, item_dim), jnp.float32),
            jax.ShapeDtypeStruct((NC, n, item_dim), jnp.float32),
            jax.ShapeDtypeStruct((NC, n, op_dim), jnp.float32),
        ),
        mesh=mesh,
        compiler_params=pltpu.CompilerParams(use_tc_tiling_on_sc=False),
        scratch_types=[
            pltpu.VMEM_SHARED((n_acc, item_dim), jnp.float32),
            pltpu.VMEM_SHARED((n_acc, op_dim), jnp.float32),
            pltpu.VMEM((KQ, CHUNK), jnp.int32),
            pltpu.VMEM((KQ, CHUNK), jnp.int32),
            pltpu.VMEM((KQ, CHUNK), jnp.int32),
            pltpu.VMEM((KQ, CHUNK), jnp.int32),
            pltpu.VMEM((KR, CHUNK, item_dim), jnp.float32),
            pltpu.VMEM((KR, CHUNK, op_dim), jnp.float32),
            pltpu.VMEM((16,), jnp.int32),
            pltpu.SemaphoreType.DMA,
            pltpu.SemaphoreType.DMA,
            pltpu.SemaphoreType.DMA,
            pltpu.SemaphoreType.DMA,
            pltpu.SemaphoreType.DMA,
            pltpu.SemaphoreType.DMA,
        ],
    )


def _tc_self_body(items_ref, Ws1, bs1, Ws2, bs2, out_ref):
    h = jnp.maximum(jnp.dot(items_ref[...], Ws1[...]) + bs1[...], 0.0)
    out_ref[...] = jnp.dot(h, Ws2[...]) + bs2[...]


def _tc_body(n, blk, self_ref, par_ref, accc_ref, acco_ref,
             Wp1, bp1, Wp2, bp2, Wc1, bc1, Wc2, bc2,
             Wo1, bo1, Wo2, bo2, Wm1, bm1, Wm2, bm2, Wm3, bm3, out_ref):
    def mlp2(x, W1, b1, W2, b2):
        h = jnp.maximum(jnp.dot(x, W1[...]) + b1[...], 0.0)
        return jnp.dot(h, W2[...]) + b2[...]

    self_emb = self_ref[...]
    parent_emb = mlp2(par_ref[...], Wp1, bp1, Wp2, bp2)
    child_in = accc_ref[0] + accc_ref[1]
    child_emb = mlp2(child_in, Wc1, bc1, Wc2, bc2)
    ops_in = acco_ref[0] + acco_ref[1]
    ops_emb = mlp2(ops_in, Wo1, bo1, Wo2, bo2)

    comb = jnp.concatenate([parent_emb, child_emb, ops_emb, self_emb], axis=-1)
    h = jnp.maximum(jnp.dot(comb, Wm1[...]) + bm1[...], 0.0)
    h = jnp.maximum(jnp.dot(h, Wm2[...]) + bm2[...], 0.0)
    h = jnp.dot(h, Wm3[...]) + bm3[...]

    i = pl.program_id(0)
    gid = i * blk + lax.broadcasted_iota(jnp.int32, h.shape, 0)
    out_ref[...] = jnp.where(gid == n - 1, 0.0, h)


def kernel(items, parents, operations, item_edge_index, op_edge_index,
           Ws1, bs1, Ws2, bs2, Wp1, bp1, Wp2, bp2, Wc1, bc1, Wc2, bc2,
           Wo1, bo1, Wo2, bo2, Wm1, bm1, Wm2, bm2, Wm3, bm3):
    n, item_dim = items.shape
    op_dim = operations.shape[1]
    e = item_edge_index.shape[1]
    out_dim = Wm3.shape[1]

    parents32 = parents.astype(jnp.int32)
    iedge = item_edge_index.astype(jnp.int32)
    oedge = op_edge_index.astype(jnp.int32)
    zitems = jnp.zeros((n, item_dim), jnp.float32)
    zops = jnp.zeros((n, op_dim), jnp.float32)

    sc = _make_sc_kernel(n, e, item_dim, op_dim)
    par_rows, accc, acco = sc(items, operations, parents32, iedge, oedge,
                              zitems, zops)

    blk = 1000
    grid = n // blk
    full = lambda shape: pl.BlockSpec(shape, lambda i: (0,) * len(shape))

    self_emb = pl.pallas_call(
        _tc_self_body,
        grid=(grid,),
        in_specs=[pl.BlockSpec((blk, item_dim), lambda i: (i, 0))]
        + [full(w.shape) for w in (Ws1, bs1, Ws2, bs2)],
        out_specs=pl.BlockSpec((blk, Ws2.shape[1]), lambda i: (i, 0)),
        out_shape=jax.ShapeDtypeStruct((n, Ws2.shape[1]), jnp.float32),
    )(items, Ws1, bs1, Ws2, bs2)

    w_specs = [full(w.shape) for w in
               (Wp1, bp1, Wp2, bp2, Wc1, bc1, Wc2, bc2,
                Wo1, bo1, Wo2, bo2, Wm1, bm1, Wm2, bm2, Wm3, bm3)]

    out = pl.pallas_call(
        functools.partial(_tc_body, n, blk),
        grid=(grid,),
        in_specs=[
            pl.BlockSpec((blk, Ws2.shape[1]), lambda i: (i, 0)),
            pl.BlockSpec((blk, item_dim), lambda i: (i, 0)),
            pl.BlockSpec((NC, blk, item_dim), lambda i: (0, i, 0)),
            pl.BlockSpec((NC, blk, op_dim), lambda i: (0, i, 0)),
        ] + w_specs,
        out_specs=pl.BlockSpec((blk, out_dim), lambda i: (i, 0)),
        out_shape=jax.ShapeDtypeStruct((n, out_dim), jnp.float32),
    )(self_emb, par_rows, accc, acco,
      Wp1, bp1, Wp2, bp2, Wc1, bc1, Wc2, bc2,
      Wo1, bo1, Wo2, bo2, Wm1, bm1, Wm2, bm2, Wm3, bm3)
    return out


# speedup vs baseline: 2.1804x; 1.0011x over previous
import functools

import jax
import jax.numpy as jnp
from jax import lax
from jax.experimental import pallas as pl
from jax.experimental.pallas import tpu as pltpu
from jax.experimental.pallas import tpu_sc as plsc

NC = 2
NS = 16
NW = NC * NS
CHUNK = 64


def _make_sc_kernel(n, e, item_dim, op_dim):
    ec = e // CHUNK
    niter_e = -(-ec // NW)
    pc = n // CHUNK
    ptail = n - pc * CHUNK
    niter_p = -(-pc // NW)
    n_acc = n + 8
    ZROWS = 400
    n_zchunks = n // ZROWS
    n_ziter = n_zchunks // NS + 1
    KQ = 8
    KR = 2

    mesh = plsc.VectorSubcoreMesh(core_axis_name="c", subcore_axis_name="s",
                                  num_cores=NC, num_subcores=NS)

    def body(items_hbm, ops_hbm, parents_hbm, iedge_hbm, oedge_hbm,
             zitems_hbm, zops_hbm,
             par_out, accc_out, acco_out,
             acc_items, acc_ops,
             si_ring, di_ring, si2_ring, di2_ring, rows, oprows, pidx_t,
             sem_i, sem_g, sem_s, sem_i2, sem_g2, sem_s2):
        cid = lax.axis_index("c")
        sid = lax.axis_index("s")
        wid = sid * NC + cid

        def zbody(k, _):
            c = sid + NS * k

            @pl.when(c < n_zchunks)
            def _():
                r0 = c * ZROWS
                pltpu.sync_copy(zitems_hbm.at[pl.ds(r0, ZROWS)],
                                acc_items.at[pl.ds(r0, ZROWS)])
                pltpu.sync_copy(zops_hbm.at[pl.ds(r0, ZROWS)],
                                acc_ops.at[pl.ds(r0, ZROWS)])
            return ()

        lax.fori_loop(0, n_ziter, zbody, (), unroll=False)
        plsc.subcore_barrier()

        def make_stream(edge_hbm, table_hbm, acc, ring, si_r, di_r,
                        s_i, s_g, s_s):
            def cond(k):
                return wid + NW * k < ec

            def start_idx(k):
                base = (wid + NW * k) * CHUNK
                q = lax.rem(k, KQ)
                pltpu.async_copy(edge_hbm.at[1, pl.ds(base, CHUNK)],
                                 si_r.at[q], s_i)
                pltpu.async_copy(edge_hbm.at[0, pl.ds(base, CHUNK)],
                                 di_r.at[q], s_i)

            def wait_idx():
                pltpu.make_async_copy(edge_hbm.at[1, pl.ds(0, CHUNK)],
                                      si_r.at[0], s_i).wait()
                pltpu.make_async_copy(edge_hbm.at[0, pl.ds(0, CHUNK)],
                                      di_r.at[0], s_i).wait()

            def start_gather(k):
                q = lax.rem(k, KQ)
                b = lax.rem(k, KR)
                pltpu.async_copy(table_hbm.at[si_r.at[q]], ring.at[b], s_g)

            def wait_gather():
                pltpu.make_async_copy(table_hbm.at[si_r.at[0]], ring.at[0],
                                      s_g).wait()

            def start_scatter(k):
                q = lax.rem(k, KQ)
                b = lax.rem(k, KR)
                pltpu.async_copy(ring.at[b], acc.at[di_r.at[q]], s_s,
                                 add=True)

            def wait_scatter():
                pltpu.make_async_copy(ring.at[0], acc.at[di_r.at[0]],
                                      s_s).wait()

            return cond, start_idx, wait_idx, start_gather, wait_gather, \
                start_scatter, wait_scatter

        icond, i_sidx, i_widx, i_sg, i_wg, i_ss, i_ws = make_stream(
            iedge_hbm, items_hbm, acc_items, rows, si_ring, di_ring,
            sem_i, sem_g, sem_s)
        ocond, o_sidx, o_widx, o_sg, o_wg, o_ss, o_ws = make_stream(
            oedge_hbm, ops_hbm, acc_ops, oprows, si2_ring, di2_ring,
            sem_i2, sem_g2, sem_s2)

        for q in range(KQ):
            i_sidx(q)
            o_sidx(q)

        def lbody(j, _):
            @pl.when(icond(j))
            def _():
                i_widx()
                o_widx()

            @pl.when(jnp.logical_and(j >= KR, icond(j - KR)))
            def _():
                i_ws()
                o_ws()

            @pl.when(jnp.logical_and(
                j >= KR, jnp.logical_and(j - KR + KQ < niter_e,
                                         icond(j - KR + KQ))))
            def _():
                i_sidx(j - KR + KQ)
                o_sidx(j - KR + KQ)

            @pl.when(icond(j))
            def _():
                i_sg(j)
                o_sg(j)

            @pl.when(jnp.logical_and(j > 0, icond(j - 1)))
            def _():
                i_wg()
                i_ss(j - 1)
                o_wg()
                o_ss(j - 1)
            return ()

        lax.fori_loop(0, niter_e, lbody, (), unroll=False)

        @pl.when(icond(niter_e - 1))
        def _():
            i_wg()
            i_ss(niter_e - 1)
            o_wg()
            o_ss(niter_e - 1)

        for t in range(niter_e - KR, niter_e):
            @pl.when(icond(t))
            def _():
                i_ws()
                o_ws()

        def pbody(k, _):
            c = wid + NW * k

            @pl.when(c < pc)
            def _():
                base = c * CHUNK
                pltpu.sync_copy(parents_hbm.at[pl.ds(base, CHUNK)],
                                si_ring.at[0])
                pltpu.async_copy(items_hbm.at[si_ring.at[0]], rows.at[0],
                                 sem_g).wait()
                pltpu.sync_copy(rows.at[0],
                                par_out.at[pl.ds(base, CHUNK)])
            return ()

        lax.fori_loop(0, niter_p, pbody, (), unroll=False)

        if ptail:
            @pl.when(wid == 0)
            def _():
                base = pc * CHUNK
                pltpu.sync_copy(parents_hbm.at[pl.ds(base, ptail)], pidx_t)
                pltpu.async_copy(items_hbm.at[pidx_t],
                                 rows.at[0, pl.ds(0, ptail)], sem_g).wait()
                pltpu.sync_copy(rows.at[0, pl.ds(0, ptail)],
                                par_out.at[pl.ds(base, ptail)])

        plsc.subcore_barrier()

        def wbody(k, _):
            c = sid + NS * k

            @pl.when(c < n_zchunks)
            def _():
                r0 = c * ZROWS
                pltpu.sync_copy(acc_items.at[pl.ds(r0, ZROWS)],
                                accc_out.at[cid, pl.ds(r0, ZROWS)])
                pltpu.sync_copy(acc_ops.at[pl.ds(r0, ZROWS)],
                                acco_out.at[cid, pl.ds(r0, ZROWS)])
            return ()

        lax.fori_loop(0, n_ziter, wbody, (), unroll=False)

    return pl.kernel(
        body,
        out_type=(
            jax.ShapeDtypeStruct((n, item_dim), jnp.float32),
            jax.ShapeDtypeStruct((NC, n, item_dim), jnp.float32),
            jax.ShapeDtypeStruct((NC, n, op_dim), jnp.float32),
        ),
        mesh=mesh,
        compiler_params=pltpu.CompilerParams(use_tc_tiling_on_sc=False),
        scratch_types=[
            pltpu.VMEM_SHARED((n_acc, item_dim), jnp.float32),
            pltpu.VMEM_SHARED((n_acc, op_dim), jnp.float32),
            pltpu.VMEM((KQ, CHUNK), jnp.int32),
            pltpu.VMEM((KQ, CHUNK), jnp.int32),
            pltpu.VMEM((KQ, CHUNK), jnp.int32),
            pltpu.VMEM((KQ, CHUNK), jnp.int32),
            pltpu.VMEM((KR, CHUNK, item_dim), jnp.float32),
            pltpu.VMEM((KR, CHUNK, op_dim), jnp.float32),
            pltpu.VMEM((16,), jnp.int32),
            pltpu.SemaphoreType.DMA,
            pltpu.SemaphoreType.DMA,
            pltpu.SemaphoreType.DMA,
            pltpu.SemaphoreType.DMA,
            pltpu.SemaphoreType.DMA,
            pltpu.SemaphoreType.DMA,
        ],
    )


def _bdot(x, w):
    return jnp.dot(x.astype(jnp.bfloat16), w.astype(jnp.bfloat16),
                   preferred_element_type=jnp.float32)


def _tc_self_body(items_ref, Ws1, bs1, Ws2, bs2, out_ref):
    h = jnp.maximum(_bdot(items_ref[...], Ws1[...]) + bs1[...], 0.0)
    out_ref[...] = _bdot(h, Ws2[...]) + bs2[...]


def _tc_body(n, blk, self_ref, par_ref, accc_ref, acco_ref,
             Wp1, bp1, Wp2, bp2, Wc1, bc1, Wc2, bc2,
             Wo1, bo1, Wo2, bo2, Wm1, bm1, Wm2, bm2, Wm3, bm3, out_ref):
    def mlp2(x, W1, b1, W2, b2):
        h = jnp.maximum(_bdot(x, W1[...]) + b1[...], 0.0)
        return _bdot(h, W2[...]) + b2[...]

    self_emb = self_ref[...]
    parent_emb = mlp2(par_ref[...], Wp1, bp1, Wp2, bp2)
    child_in = accc_ref[0] + accc_ref[1]
    child_emb = mlp2(child_in, Wc1, bc1, Wc2, bc2)
    ops_in = acco_ref[0] + acco_ref[1]
    ops_emb = mlp2(ops_in, Wo1, bo1, Wo2, bo2)

    comb = jnp.concatenate([parent_emb, child_emb, ops_emb, self_emb], axis=-1)
    h = jnp.maximum(_bdot(comb, Wm1[...]) + bm1[...], 0.0)
    h = jnp.maximum(_bdot(h, Wm2[...]) + bm2[...], 0.0)
    h = _bdot(h, Wm3[...]) + bm3[...]

    i = pl.program_id(0)
    gid = i * blk + lax.broadcasted_iota(jnp.int32, h.shape, 0)
    out_ref[...] = jnp.where(gid == n - 1, 0.0, h)


def kernel(items, parents, operations, item_edge_index, op_edge_index,
           Ws1, bs1, Ws2, bs2, Wp1, bp1, Wp2, bp2, Wc1, bc1, Wc2, bc2,
           Wo1, bo1, Wo2, bo2, Wm1, bm1, Wm2, bm2, Wm3, bm3):
    n, item_dim = items.shape
    op_dim = operations.shape[1]
    e = item_edge_index.shape[1]
    out_dim = Wm3.shape[1]

    parents32 = parents.astype(jnp.int32)
    iedge = item_edge_index.astype(jnp.int32)
    oedge = op_edge_index.astype(jnp.int32)
    zitems = jnp.zeros((n, item_dim), jnp.float32)
    zops = jnp.zeros((n, op_dim), jnp.float32)

    sc = _make_sc_kernel(n, e, item_dim, op_dim)
    par_rows, accc, acco = sc(items, operations, parents32, iedge, oedge,
                              zitems, zops)

    blk = 1000
    grid = n // blk
    full = lambda shape: pl.BlockSpec(shape, lambda i: (0,) * len(shape))

    self_emb = pl.pallas_call(
        _tc_self_body,
        grid=(grid,),
        in_specs=[pl.BlockSpec((blk, item_dim), lambda i: (i, 0))]
        + [full(w.shape) for w in (Ws1, bs1, Ws2, bs2)],
        out_specs=pl.BlockSpec((blk, Ws2.shape[1]), lambda i: (i, 0)),
        out_shape=jax.ShapeDtypeStruct((n, Ws2.shape[1]), jnp.float32),
    )(items, Ws1, bs1, Ws2, bs2)

    w_specs = [full(w.shape) for w in
               (Wp1, bp1, Wp2, bp2, Wc1, bc1, Wc2, bc2,
                Wo1, bo1, Wo2, bo2, Wm1, bm1, Wm2, bm2, Wm3, bm3)]

    out = pl.pallas_call(
        functools.partial(_tc_body, n, blk),
        grid=(grid,),
        in_specs=[
            pl.BlockSpec((blk, Ws2.shape[1]), lambda i: (i, 0)),
            pl.BlockSpec((blk, item_dim), lambda i: (i, 0)),
            pl.BlockSpec((NC, blk, item_dim), lambda i: (0, i, 0)),
            pl.BlockSpec((NC, blk, op_dim), lambda i: (0, i, 0)),
        ] + w_specs,
        out_specs=pl.BlockSpec((blk, out_dim), lambda i: (i, 0)),
        out_shape=jax.ShapeDtypeStruct((n, out_dim), jnp.float32),
    )(self_emb, par_rows, accc, acco,
      Wp1, bp1, Wp2, bp2, Wc1, bc1, Wc2, bc2,
      Wo1, bo1, Wo2, bo2, Wm1, bm1, Wm2, bm2, Wm3, bm3)
    return out


# speedup vs baseline: 2.1812x; 1.0003x over previous
import functools

import jax
import jax.numpy as jnp
from jax import lax
from jax.experimental import pallas as pl
from jax.experimental.pallas import tpu as pltpu
from jax.experimental.pallas import tpu_sc as plsc

NC = 2
NS = 16
NW = NC * NS
CHUNK = 64


def _make_sc_kernel(n, e, item_dim, op_dim):
    ec = e // CHUNK
    niter_e = -(-ec // NW)
    pc = n // CHUNK
    ptail = n - pc * CHUNK
    niter_p = -(-pc // NW)
    n_acc = n + 8
    ZROWS = 400
    n_zchunks = n // ZROWS
    n_ziter = n_zchunks // NS + 1
    KQ = 8
    KR = 2

    mesh = plsc.VectorSubcoreMesh(core_axis_name="c", subcore_axis_name="s",
                                  num_cores=NC, num_subcores=NS)

    def body(items_hbm, ops_hbm, parents_hbm, iedge_hbm, oedge_hbm,
             zitems_hbm, zops_hbm,
             par_out, accc_out, acco_out,
             acc_items, acc_ops,
             si_ring, di_ring, si2_ring, di2_ring, rows, oprows, pidx_t,
             sem_i, sem_g, sem_s, sem_i2, sem_g2, sem_s2):
        cid = lax.axis_index("c")
        sid = lax.axis_index("s")
        wid = sid * NC + cid

        def zbody(k, _):
            c = sid + NS * k

            @pl.when(c < n_zchunks)
            def _():
                r0 = c * ZROWS
                pltpu.sync_copy(zitems_hbm.at[pl.ds(r0, ZROWS)],
                                acc_items.at[pl.ds(r0, ZROWS)])
                pltpu.sync_copy(zops_hbm.at[pl.ds(r0, ZROWS)],
                                acc_ops.at[pl.ds(r0, ZROWS)])
            return ()

        lax.fori_loop(0, n_ziter, zbody, (), unroll=False)
        plsc.subcore_barrier()

        def make_stream(edge_hbm, table_hbm, acc, ring, si_r, di_r,
                        s_i, s_g, s_s):
            def cond(k):
                return wid + NW * k < ec

            def start_idx(k):
                base = (wid + NW * k) * CHUNK
                q = lax.rem(k, KQ)
                pltpu.async_copy(edge_hbm.at[1, pl.ds(base, CHUNK)],
                                 si_r.at[q], s_i)
                pltpu.async_copy(edge_hbm.at[0, pl.ds(base, CHUNK)],
                                 di_r.at[q], s_i)

            def wait_idx():
                pltpu.make_async_copy(edge_hbm.at[1, pl.ds(0, CHUNK)],
                                      si_r.at[0], s_i).wait()
                pltpu.make_async_copy(edge_hbm.at[0, pl.ds(0, CHUNK)],
                                      di_r.at[0], s_i).wait()

            def start_gather(k):
                q = lax.rem(k, KQ)
                b = lax.rem(k, KR)
                pltpu.async_copy(table_hbm.at[si_r.at[q]], ring.at[b], s_g)

            def wait_gather():
                pltpu.make_async_copy(table_hbm.at[si_r.at[0]], ring.at[0],
                                      s_g).wait()

            def start_scatter(k):
                q = lax.rem(k, KQ)
                b = lax.rem(k, KR)
                pltpu.async_copy(ring.at[b], acc.at[di_r.at[q]], s_s,
                                 add=True)

            def wait_scatter():
                pltpu.make_async_copy(ring.at[0], acc.at[di_r.at[0]],
                                      s_s).wait()

            return cond, start_idx, wait_idx, start_gather, wait_gather, \
                start_scatter, wait_scatter

        icond, i_sidx, i_widx, i_sg, i_wg, i_ss, i_ws = make_stream(
            iedge_hbm, items_hbm, acc_items, rows, si_ring, di_ring,
            sem_i, sem_g, sem_s)
        ocond, o_sidx, o_widx, o_sg, o_wg, o_ss, o_ws = make_stream(
            oedge_hbm, ops_hbm, acc_ops, oprows, si2_ring, di2_ring,
            sem_i2, sem_g2, sem_s2)

        for q in range(KQ):
            i_sidx(q)
            o_sidx(q)

        def lbody(j, _):
            @pl.when(icond(j))
            def _():
                i_widx()
                o_widx()

            @pl.when(jnp.logical_and(j >= KR, icond(j - KR)))
            def _():
                i_ws()
                o_ws()

            @pl.when(jnp.logical_and(
                j >= KR, jnp.logical_and(j - KR + KQ < niter_e,
                                         icond(j - KR + KQ))))
            def _():
                i_sidx(j - KR + KQ)
                o_sidx(j - KR + KQ)

            @pl.when(icond(j))
            def _():
                i_sg(j)
                o_sg(j)

            @pl.when(jnp.logical_and(j > 0, icond(j - 1)))
            def _():
                i_wg()
                i_ss(j - 1)
                o_wg()
                o_ss(j - 1)
            return ()

        lax.fori_loop(0, niter_e, lbody, (), unroll=False)

        @pl.when(icond(niter_e - 1))
        def _():
            i_wg()
            i_ss(niter_e - 1)
            o_wg()
            o_ss(niter_e - 1)

        for t in range(niter_e - KR, niter_e):
            @pl.when(icond(t))
            def _():
                i_ws()
                o_ws()

        def pbody(k, _):
            c = wid + NW * k

            @pl.when(c < pc)
            def _():
                base = c * CHUNK
                pltpu.sync_copy(parents_hbm.at[pl.ds(base, CHUNK)],
                                si_ring.at[0])
                pltpu.async_copy(items_hbm.at[si_ring.at[0]], rows.at[0],
                                 sem_g).wait()
                pltpu.sync_copy(rows.at[0],
                                par_out.at[pl.ds(base, CHUNK)])
            return ()

        lax.fori_loop(0, niter_p, pbody, (), unroll=False)

        if ptail:
            @pl.when(wid == 0)
            def _():
                base = pc * CHUNK
                pltpu.sync_copy(parents_hbm.at[pl.ds(base, ptail)], pidx_t)
                pltpu.async_copy(items_hbm.at[pidx_t],
                                 rows.at[0, pl.ds(0, ptail)], sem_g).wait()
                pltpu.sync_copy(rows.at[0, pl.ds(0, ptail)],
                                par_out.at[pl.ds(base, ptail)])

        plsc.subcore_barrier()

        def wbody(k, _):
            c = sid + NS * k

            @pl.when(c < n_zchunks)
            def _():
                r0 = c * ZROWS
                pltpu.sync_copy(acc_items.at[pl.ds(r0, ZROWS)],
                                accc_out.at[cid, pl.ds(r0, ZROWS)])
                pltpu.sync_copy(acc_ops.at[pl.ds(r0, ZROWS)],
                                acco_out.at[cid, pl.ds(r0, ZROWS)])
            return ()

        lax.fori_loop(0, n_ziter, wbody, (), unroll=False)

    return pl.kernel(
        body,
        out_type=(
            jax.ShapeDtypeStruct((n, item_dim), jnp.float32),
            jax.ShapeDtypeStruct((NC, n, item_dim), jnp.float32),
            jax.ShapeDtypeStruct((NC, n, op_dim), jnp.float32),
        ),
        mesh=mesh,
        compiler_params=pltpu.CompilerParams(use_tc_tiling_on_sc=False),
        scratch_types=[
            pltpu.VMEM_SHARED((n_acc, item_dim), jnp.float32),
            pltpu.VMEM_SHARED((n_acc, op_dim), jnp.float32),
            pltpu.VMEM((KQ, CHUNK), jnp.int32),
            pltpu.VMEM((KQ, CHUNK), jnp.int32),
            pltpu.VMEM((KQ, CHUNK), jnp.int32),
            pltpu.VMEM((KQ, CHUNK), jnp.int32),
            pltpu.VMEM((KR, CHUNK, item_dim), jnp.float32),
            pltpu.VMEM((KR, CHUNK, op_dim), jnp.float32),
            pltpu.VMEM((16,), jnp.int32),
            pltpu.SemaphoreType.DMA,
            pltpu.SemaphoreType.DMA,
            pltpu.SemaphoreType.DMA,
            pltpu.SemaphoreType.DMA,
            pltpu.SemaphoreType.DMA,
            pltpu.SemaphoreType.DMA,
        ],
    )


def _tc_self_body(items_ref, Ws1, bs1, Ws2, bs2, out_ref):
    h = jnp.maximum(jnp.dot(items_ref[...], Ws1[...]) + bs1[...], 0.0)
    out_ref[...] = jnp.dot(h, Ws2[...]) + bs2[...]


def _tc_body(n, blk, self_ref, par_ref, accc_ref, acco_ref,
             Wp1, bp1, Wp2, bp2, Wc1, bc1, Wc2, bc2,
             Wo1, bo1, Wo2, bo2, Wm1, bm1, Wm2, bm2, Wm3, bm3, out_ref):
    def mlp2(x, W1, b1, W2, b2):
        h = jnp.maximum(jnp.dot(x, W1[...]) + b1[...], 0.0)
        return jnp.dot(h, W2[...]) + b2[...]

    self_emb = self_ref[...]
    parent_emb = mlp2(par_ref[...], Wp1, bp1, Wp2, bp2)
    child_in = accc_ref[0] + accc_ref[1]
    child_emb = mlp2(child_in, Wc1, bc1, Wc2, bc2)
    ops_in = acco_ref[0] + acco_ref[1]
    ops_emb = mlp2(ops_in, Wo1, bo1, Wo2, bo2)

    comb = jnp.concatenate([parent_emb, child_emb, ops_emb, self_emb], axis=-1)
    h = jnp.maximum(jnp.dot(comb, Wm1[...]) + bm1[...], 0.0)
    h = jnp.maximum(jnp.dot(h, Wm2[...]) + bm2[...], 0.0)
    h = jnp.dot(h, Wm3[...]) + bm3[...]

    i = pl.program_id(0)
    gid = i * blk + lax.broadcasted_iota(jnp.int32, h.shape, 0)
    out_ref[...] = jnp.where(gid == n - 1, 0.0, h)


def kernel(items, parents, operations, item_edge_index, op_edge_index,
           Ws1, bs1, Ws2, bs2, Wp1, bp1, Wp2, bp2, Wc1, bc1, Wc2, bc2,
           Wo1, bo1, Wo2, bo2, Wm1, bm1, Wm2, bm2, Wm3, bm3):
    n, item_dim = items.shape
    op_dim = operations.shape[1]
    e = item_edge_index.shape[1]
    out_dim = Wm3.shape[1]

    parents32 = parents.astype(jnp.int32)
    iedge = item_edge_index.astype(jnp.int32)
    oedge = op_edge_index.astype(jnp.int32)
    zitems = jnp.zeros((n, item_dim), jnp.float32)
    zops = jnp.zeros((n, op_dim), jnp.float32)

    sc = _make_sc_kernel(n, e, item_dim, op_dim)
    par_rows, accc, acco = sc(items, operations, parents32, iedge, oedge,
                              zitems, zops)

    blk = 1000
    grid = n // blk
    full = lambda shape: pl.BlockSpec(shape, lambda i: (0,) * len(shape))

    self_emb = pl.pallas_call(
        _tc_self_body,
        grid=(grid,),
        in_specs=[pl.BlockSpec((blk, item_dim), lambda i: (i, 0))]
        + [full(w.shape) for w in (Ws1, bs1, Ws2, bs2)],
        out_specs=pl.BlockSpec((blk, Ws2.shape[1]), lambda i: (i, 0)),
        out_shape=jax.ShapeDtypeStruct((n, Ws2.shape[1]), jnp.float32),
    )(items, Ws1, bs1, Ws2, bs2)

    w_specs = [full(w.shape) for w in
               (Wp1, bp1, Wp2, bp2, Wc1, bc1, Wc2, bc2,
                Wo1, bo1, Wo2, bo2, Wm1, bm1, Wm2, bm2, Wm3, bm3)]

    out = pl.pallas_call(
        functools.partial(_tc_body, n, blk),
        grid=(grid,),
        in_specs=[
            pl.BlockSpec((blk, Ws2.shape[1]), lambda i: (i, 0)),
            pl.BlockSpec((blk, item_dim), lambda i: (i, 0)),
            pl.BlockSpec((NC, blk, item_dim), lambda i: (0, i, 0)),
            pl.BlockSpec((NC, blk, op_dim), lambda i: (0, i, 0)),
        ] + w_specs,
        out_specs=pl.BlockSpec((blk, out_dim), lambda i: (i, 0)),
        out_shape=jax.ShapeDtypeStruct((n, out_dim), jnp.float32),
    )(self_emb, par_rows, accc, acco,
      Wp1, bp1, Wp2, bp2, Wc1, bc1, Wc2, bc2,
      Wo1, bo1, Wo2, bo2, Wm1, bm1, Wm2, bm2, Wm3, bm3)
    return out


# speedup vs baseline: 2.2445x; 1.0290x over previous
import functools

import jax
import jax.numpy as jnp
from jax import lax
from jax.experimental import pallas as pl
from jax.experimental.pallas import tpu as pltpu
from jax.experimental.pallas import tpu_sc as plsc

NC = 2
NS = 16
NW = NC * NS
CHUNK = 64


def _make_sc_kernel(n, e, item_dim, op_dim):
    ec = e // CHUNK
    niter_e = -(-ec // NW)
    pc = n // CHUNK
    ptail = n - pc * CHUNK
    niter_p = -(-pc // NW)
    n_acc = n + 8
    ZROWS = 400
    n_zchunks = n // ZROWS
    n_ziter = n_zchunks // NS + 1
    KQ = 8
    KR = 2

    mesh = plsc.VectorSubcoreMesh(core_axis_name="c", subcore_axis_name="s",
                                  num_cores=NC, num_subcores=NS)

    def body(items_hbm, ops_hbm, parents_hbm, iedge_hbm, oedge_hbm,
             zitems_hbm, zops_hbm,
             par_out, accc_out, acco_out,
             acc_items, acc_ops,
             si_ring, di_ring, si2_ring, di2_ring, rows, oprows, pidx_t,
             pidx_ring, prows,
             sem_i, sem_g, sem_s, sem_i2, sem_g2, sem_s2,
             sem_pi, sem_pg, sem_pw):
        cid = lax.axis_index("c")
        sid = lax.axis_index("s")
        wid = sid * NC + cid

        def zbody(k, _):
            c = sid + NS * k

            @pl.when(c < n_zchunks)
            def _():
                r0 = c * ZROWS
                pltpu.sync_copy(zitems_hbm.at[pl.ds(r0, ZROWS)],
                                acc_items.at[pl.ds(r0, ZROWS)])
                pltpu.sync_copy(zops_hbm.at[pl.ds(r0, ZROWS)],
                                acc_ops.at[pl.ds(r0, ZROWS)])
            return ()

        lax.fori_loop(0, n_ziter, zbody, (), unroll=False)
        plsc.subcore_barrier()

        def make_stream(edge_hbm, table_hbm, acc, ring, si_r, di_r,
                        s_i, s_g, s_s):
            def cond(k):
                return wid + NW * k < ec

            def start_idx(k):
                base = (wid + NW * k) * CHUNK
                q = lax.rem(k, KQ)
                pltpu.async_copy(edge_hbm.at[1, pl.ds(base, CHUNK)],
                                 si_r.at[q], s_i)
                pltpu.async_copy(edge_hbm.at[0, pl.ds(base, CHUNK)],
                                 di_r.at[q], s_i)

            def wait_idx():
                pltpu.make_async_copy(edge_hbm.at[1, pl.ds(0, CHUNK)],
                                      si_r.at[0], s_i).wait()
                pltpu.make_async_copy(edge_hbm.at[0, pl.ds(0, CHUNK)],
                                      di_r.at[0], s_i).wait()

            def start_gather(k):
                q = lax.rem(k, KQ)
                b = lax.rem(k, KR)
                pltpu.async_copy(table_hbm.at[si_r.at[q]], ring.at[b], s_g)

            def wait_gather():
                pltpu.make_async_copy(table_hbm.at[si_r.at[0]], ring.at[0],
                                      s_g).wait()

            def start_scatter(k):
                q = lax.rem(k, KQ)
                b = lax.rem(k, KR)
                pltpu.async_copy(ring.at[b], acc.at[di_r.at[q]], s_s,
                                 add=True)

            def wait_scatter():
                pltpu.make_async_copy(ring.at[0], acc.at[di_r.at[0]],
                                      s_s).wait()

            return cond, start_idx, wait_idx, start_gather, wait_gather, \
                start_scatter, wait_scatter

        icond, i_sidx, i_widx, i_sg, i_wg, i_ss, i_ws = make_stream(
            iedge_hbm, items_hbm, acc_items, rows, si_ring, di_ring,
            sem_i, sem_g, sem_s)
        ocond, o_sidx, o_widx, o_sg, o_wg, o_ss, o_ws = make_stream(
            oedge_hbm, ops_hbm, acc_ops, oprows, si2_ring, di2_ring,
            sem_i2, sem_g2, sem_s2)

        def p_cond(k):
            return wid + NW * k < pc

        def p_base(k):
            return (wid + NW * k) * CHUNK

        def p_sidx(k):
            pltpu.async_copy(parents_hbm.at[pl.ds(p_base(k), CHUNK)],
                             pidx_ring.at[k], sem_pi)

        def p_widx():
            pltpu.make_async_copy(parents_hbm.at[pl.ds(0, CHUNK)],
                                  pidx_ring.at[0], sem_pi).wait()

        def p_sg(k):
            pltpu.async_copy(items_hbm.at[pidx_ring.at[k]],
                             prows.at[lax.rem(k, 2)], sem_pg)

        def p_wg():
            pltpu.make_async_copy(items_hbm.at[pidx_ring.at[0]],
                                  prows.at[0], sem_pg).wait()

        def p_sw(k):
            pltpu.async_copy(prows.at[lax.rem(k, 2)],
                             par_out.at[pl.ds(p_base(k), CHUNK)], sem_pw)

        def p_ww():
            pltpu.make_async_copy(prows.at[0],
                                  par_out.at[pl.ds(0, CHUNK)], sem_pw).wait()

        for q in range(KQ):
            i_sidx(q)
            o_sidx(q)
        for k in range(niter_p):
            @pl.when(p_cond(k))
            def _():
                p_sidx(k)

        def lbody(j, _):
            @pl.when(icond(j))
            def _():
                i_widx()
                o_widx()

            @pl.when(jnp.logical_and(j >= KR, icond(j - KR)))
            def _():
                i_ws()
                o_ws()

            @pl.when(jnp.logical_and(
                j >= KR, jnp.logical_and(j - KR + KQ < niter_e,
                                         icond(j - KR + KQ))))
            def _():
                i_sidx(j - KR + KQ)
                o_sidx(j - KR + KQ)

            @pl.when(icond(j))
            def _():
                i_sg(j)
                o_sg(j)

            @pl.when(jnp.logical_and(j > 0, icond(j - 1)))
            def _():
                i_wg()
                i_ss(j - 1)
                o_wg()
                o_ss(j - 1)

            @pl.when(jnp.logical_and(j < niter_p, p_cond(j)))
            def _():
                p_widx()
                p_sg(j)

            @pl.when(jnp.logical_and(
                jnp.logical_and(j >= 1, j - 1 < niter_p), p_cond(j - 1)))
            def _():
                p_wg()
                p_sw(j - 1)

            @pl.when(jnp.logical_and(
                jnp.logical_and(j >= 2, j - 2 < niter_p), p_cond(j - 2)))
            def _():
                p_ww()
            return ()

        lax.fori_loop(0, niter_e, lbody, (), unroll=False)

        @pl.when(icond(niter_e - 1))
        def _():
            i_wg()
            i_ss(niter_e - 1)
            o_wg()
            o_ss(niter_e - 1)

        for t in range(niter_e - KR, niter_e):
            @pl.when(icond(t))
            def _():
                i_ws()
                o_ws()

        if ptail:
            @pl.when(wid == 0)
            def _():
                base = pc * CHUNK
                pltpu.sync_copy(parents_hbm.at[pl.ds(base, ptail)], pidx_t)
                pltpu.async_copy(items_hbm.at[pidx_t],
                                 rows.at[0, pl.ds(0, ptail)], sem_g).wait()
                pltpu.sync_copy(rows.at[0, pl.ds(0, ptail)],
                                par_out.at[pl.ds(base, ptail)])

        plsc.subcore_barrier()

        def wbody(k, _):
            c = sid + NS * k

            @pl.when(c < n_zchunks)
            def _():
                r0 = c * ZROWS
                pltpu.sync_copy(acc_items.at[pl.ds(r0, ZROWS)],
                                accc_out.at[cid, pl.ds(r0, ZROWS)])
                pltpu.sync_copy(acc_ops.at[pl.ds(r0, ZROWS)],
                                acco_out.at[cid, pl.ds(r0, ZROWS)])
            return ()

        lax.fori_loop(0, n_ziter, wbody, (), unroll=False)

    return pl.kernel(
        body,
        out_type=(
            jax.ShapeDtypeStruct((n, item_dim), jnp.float32),
            jax.ShapeDtypeStruct((NC, n, item_dim), jnp.float32),
            jax.ShapeDtypeStruct((NC, n, op_dim), jnp.float32),
        ),
        mesh=mesh,
        compiler_params=pltpu.CompilerParams(use_tc_tiling_on_sc=False),
        scratch_types=[
            pltpu.VMEM_SHARED((n_acc, item_dim), jnp.float32),
            pltpu.VMEM_SHARED((n_acc, op_dim), jnp.float32),
            pltpu.VMEM((KQ, CHUNK), jnp.int32),
            pltpu.VMEM((KQ, CHUNK), jnp.int32),
            pltpu.VMEM((KQ, CHUNK), jnp.int32),
            pltpu.VMEM((KQ, CHUNK), jnp.int32),
            pltpu.VMEM((KR, CHUNK, item_dim), jnp.float32),
            pltpu.VMEM((KR, CHUNK, op_dim), jnp.float32),
            pltpu.VMEM((16,), jnp.int32),
            pltpu.VMEM((niter_p, CHUNK), jnp.int32),
            pltpu.VMEM((2, CHUNK, item_dim), jnp.float32),
            pltpu.SemaphoreType.DMA,
            pltpu.SemaphoreType.DMA,
            pltpu.SemaphoreType.DMA,
            pltpu.SemaphoreType.DMA,
            pltpu.SemaphoreType.DMA,
            pltpu.SemaphoreType.DMA,
            pltpu.SemaphoreType.DMA,
            pltpu.SemaphoreType.DMA,
            pltpu.SemaphoreType.DMA,
        ],
    )


def _tc_self_body(items_ref, Ws1, bs1, Ws2, bs2, out_ref):
    h = jnp.maximum(jnp.dot(items_ref[...], Ws1[...]) + bs1[...], 0.0)
    out_ref[...] = jnp.dot(h, Ws2[...]) + bs2[...]


def _tc_body(n, blk, self_ref, par_ref, accc_ref, acco_ref,
             Wp1, bp1, Wp2, bp2, Wc1, bc1, Wc2, bc2,
             Wo1, bo1, Wo2, bo2, Wm1, bm1, Wm2, bm2, Wm3, bm3, out_ref):
    def mlp2(x, W1, b1, W2, b2):
        h = jnp.maximum(jnp.dot(x, W1[...]) + b1[...], 0.0)
        return jnp.dot(h, W2[...]) + b2[...]

    self_emb = self_ref[...]
    parent_emb = mlp2(par_ref[...], Wp1, bp1, Wp2, bp2)
    child_in = accc_ref[0] + accc_ref[1]
    child_emb = mlp2(child_in, Wc1, bc1, Wc2, bc2)
    ops_in = acco_ref[0] + acco_ref[1]
    ops_emb = mlp2(ops_in, Wo1, bo1, Wo2, bo2)

    comb = jnp.concatenate([parent_emb, child_emb, ops_emb, self_emb], axis=-1)
    h = jnp.maximum(jnp.dot(comb, Wm1[...]) + bm1[...], 0.0)
    h = jnp.maximum(jnp.dot(h, Wm2[...]) + bm2[...], 0.0)
    h = jnp.dot(h, Wm3[...]) + bm3[...]

    i = pl.program_id(0)
    gid = i * blk + lax.broadcasted_iota(jnp.int32, h.shape, 0)
    out_ref[...] = jnp.where(gid == n - 1, 0.0, h)


def kernel(items, parents, operations, item_edge_index, op_edge_index,
           Ws1, bs1, Ws2, bs2, Wp1, bp1, Wp2, bp2, Wc1, bc1, Wc2, bc2,
           Wo1, bo1, Wo2, bo2, Wm1, bm1, Wm2, bm2, Wm3, bm3):
    n, item_dim = items.shape
    op_dim = operations.shape[1]
    e = item_edge_index.shape[1]
    out_dim = Wm3.shape[1]

    parents32 = parents.astype(jnp.int32)
    iedge = item_edge_index.astype(jnp.int32)
    oedge = op_edge_index.astype(jnp.int32)
    zitems = jnp.zeros((n, item_dim), jnp.float32)
    zops = jnp.zeros((n, op_dim), jnp.float32)

    sc = _make_sc_kernel(n, e, item_dim, op_dim)
    par_rows, accc, acco = sc(items, operations, parents32, iedge, oedge,
                              zitems, zops)

    blk = 1000
    grid = n // blk
    full = lambda shape: pl.BlockSpec(shape, lambda i: (0,) * len(shape))

    self_emb = pl.pallas_call(
        _tc_self_body,
        grid=(grid,),
        in_specs=[pl.BlockSpec((blk, item_dim), lambda i: (i, 0))]
        + [full(w.shape) for w in (Ws1, bs1, Ws2, bs2)],
        out_specs=pl.BlockSpec((blk, Ws2.shape[1]), lambda i: (i, 0)),
        out_shape=jax.ShapeDtypeStruct((n, Ws2.shape[1]), jnp.float32),
    )(items, Ws1, bs1, Ws2, bs2)

    w_specs = [full(w.shape) for w in
               (Wp1, bp1, Wp2, bp2, Wc1, bc1, Wc2, bc2,
                Wo1, bo1, Wo2, bo2, Wm1, bm1, Wm2, bm2, Wm3, bm3)]

    out = pl.pallas_call(
        functools.partial(_tc_body, n, blk),
        grid=(grid,),
        in_specs=[
            pl.BlockSpec((blk, Ws2.shape[1]), lambda i: (i, 0)),
            pl.BlockSpec((blk, item_dim), lambda i: (i, 0)),
            pl.BlockSpec((NC, blk, item_dim), lambda i: (0, i, 0)),
            pl.BlockSpec((NC, blk, op_dim), lambda i: (0, i, 0)),
        ] + w_specs,
        out_specs=pl.BlockSpec((blk, out_dim), lambda i: (i, 0)),
        out_shape=jax.ShapeDtypeStruct((n, out_dim), jnp.float32),
    )(self_emb, par_rows, accc, acco,
      Wp1, bp1, Wp2, bp2, Wc1, bc1, Wc2, bc2,
      Wo1, bo1, Wo2, bo2, Wm1, bm1, Wm2, bm2, Wm3, bm3)
    return out
